# baseline (speedup 1.0000x reference)
import functools

import jax
import jax.numpy as jnp
from jax import lax
from jax.experimental import pallas as pl
from jax.experimental.pallas import tpu as pltpu
from jax.experimental.pallas import tpu_sc as plsc

N_ENTITIES = 1000000
N_RELATIONS = 1000
EMBED_DIM = 64
BATCH = 16384

NC = 2
NS = 16
NW = NC * NS
NLANE = 16
B_PER_W = BATCH // NW
CHUNK = 8
NCH = B_PER_W // CHUNK
TILES_R = N_RELATIONS // 8
RSTG = 5


def _sc_kernel(half, heads_hbm, rels_hbm, tails_hbm, ent_hbm, rel_hbm,
               out_hbm, hidx, ridx, tidx, htile, ttile, hsub, tsub,
               rtab, rstg, ebufs, tbufs, outb, sems):
    wid = lax.axis_index("s")
    base = (half * NS + wid) * B_PER_W
    obase = wid * B_PER_W

    ent_v = ent_hbm.reshape(N_ENTITIES // 8, 8, EMBED_DIM)
    rel_v = rel_hbm.reshape(TILES_R, 8, EMBED_DIM)

    pltpu.sync_copy(heads_hbm.at[pl.ds(base, B_PER_W)], hidx)
    pltpu.sync_copy(rels_hbm.at[pl.ds(base, B_PER_W)], ridx)
    pltpu.sync_copy(tails_hbm.at[pl.ds(base, B_PER_W)], tidx)
    for g in range(B_PER_W // NLANE):
        sl = pl.ds(g * NLANE, NLANE)
        for raw, tile, sub in ((hidx, htile, hsub), (tidx, ttile, tsub)):
            v = raw[sl]
            tile[sl] = lax.shift_right_logical(v, 3)
            sub[sl] = lax.bitwise_and(v, 7)

    def rel_part(p, carry):
        pltpu.sync_copy(rel_v.at[pl.ds(p * RSTG, RSTG)], rstg)

        def rel_tile(t, c2):
            for s in range(8):
                for c in range(EMBED_DIM // NLANE):
                    sl = pl.ds(c * NLANE, NLANE)
                    row = (p * RSTG + t) * 4 + s // 2
                    rtab[row, pl.ds((s % 2) * EMBED_DIM + c * NLANE, NLANE)] = (
                        rstg[t, s, sl])
            return c2

        lax.fori_loop(0, RSTG, rel_tile, 0)
        return carry

    lax.fori_loop(0, TILES_R // RSTG, rel_part, 0)

    lane = lax.iota(jnp.int32, NLANE)

    def fire_chunk(k, par, slot):
        kk = jnp.minimum(k, NCH - 1)
        goff = lax.shift_right_logical(kk, 1) * NLANE
        thv = htile[pl.ds(goff, NLANE)]
        ttv = ttile[pl.ds(goff, NLANE)]
        copies = []
        for i in range(CHUNK):
            ln = par * CHUNK + i
            copies.append(pltpu.async_copy(
                ent_v.at[thv[ln]], ebufs[slot].at[i], sems[slot]))
            copies.append(pltpu.async_copy(
                ent_v.at[ttv[ln]], tbufs[slot].at[i], sems[slot]))
        return copies

    def wait(slot):
        for i in range(CHUNK):
            pltpu.make_async_copy(ent_v.at[0], ebufs[slot].at[i],
                                  sems[slot]).wait()
            pltpu.make_async_copy(ent_v.at[0], tbufs[slot].at[i],
                                  sems[slot]).wait()

    def compute(k, par, slot, out16):
        eb, tb = ebufs[slot], tbufs[slot]
        goff = lax.shift_right_logical(k, 1) * NLANE
        gsl = pl.ds(goff, NLANE)
        hsv, tsv, rv = hsub[gsl], tsub[gsl], ridx[gsl]
        for i in range(CHUNK):
            ln = par * CHUNK + i
            sh, st, rid = hsv[ln], tsv[ln], rv[ln]
            rrow = lax.shift_right_logical(rid, 1)
            rcol = lax.bitwise_and(rid, 1) * EMBED_DIM
            acc = None
            for c in range(EMBED_DIM // NLANE):
                sl = pl.ds(c * NLANE, NLANE)
                v = (eb[i, sh, sl]
                     + rtab[rrow, pl.ds(rcol + c * NLANE, NLANE)]
                     - tb[i, st, sl])
                acc = v * v if acc is None else acc + v * v
            out16 = jnp.where(lane == ln, -jnp.sum(acc), out16)
        return out16

    fire_chunk(0, 0, 0)

    def body(g, carry):
        out16 = jnp.zeros((NLANE,), jnp.float32)
        for b in range(2):
            k = g * 2 + b
            fire_chunk(k + 1, (b + 1) % 2, 1 - b)
            wait(b)
            out16 = compute(k, b, b, out16)
        outb[pl.ds(g * NLANE, NLANE)] = out16
        return carry

    lax.fori_loop(0, NCH // 2, body, 0)
    wait(0)

    pltpu.sync_copy(outb, out_hbm.at[pl.ds(obase, B_PER_W)])


_MESH = plsc.VectorSubcoreMesh(core_axis_name="c", subcore_axis_name="s",
                               num_cores=1)
_PARAMS = pltpu.CompilerParams(needs_layout_passes=False,
                               use_tc_tiling_on_sc=True)


def _make_half(half):
    return functools.partial(
        pl.kernel,
        mesh=_MESH,
        out_type=jax.ShapeDtypeStruct((BATCH // 2,), jnp.float32),
        compiler_params=_PARAMS,
        scratch_types=[
            pltpu.VMEM((B_PER_W,), jnp.int32),
            pltpu.VMEM((B_PER_W,), jnp.int32),
            pltpu.VMEM((B_PER_W,), jnp.int32),
            pltpu.VMEM((B_PER_W,), jnp.int32),
            pltpu.VMEM((B_PER_W,), jnp.int32),
            pltpu.VMEM((B_PER_W,), jnp.int32),
            pltpu.VMEM((B_PER_W,), jnp.int32),
            pltpu.VMEM((N_RELATIONS // 2, 2 * EMBED_DIM), jnp.float32),
            pltpu.VMEM((RSTG, 8, EMBED_DIM), jnp.float32),
            [pltpu.VMEM((CHUNK, 8, EMBED_DIM), jnp.float32) for _ in range(2)],
            [pltpu.VMEM((CHUNK, 8, EMBED_DIM), jnp.float32) for _ in range(2)],
            pltpu.VMEM((B_PER_W,), jnp.float32),
            [pltpu.SemaphoreType.DMA for _ in range(2)],
        ],
    )(functools.partial(_sc_kernel, half))


@jax.jit
def kernel(heads, relations, tails, entity_embedding, relation_embedding):
    o0 = _make_half(0)(heads, relations, tails, entity_embedding,
                       relation_embedding)
    o1 = _make_half(1)(heads, relations, tails, entity_embedding,
                       relation_embedding)
    return jnp.concatenate([o0, o1])


# speedup vs baseline: 1.2685x; 1.2685x over previous
import functools

import jax
import jax.numpy as jnp
from jax import lax
from jax.experimental import pallas as pl
from jax.experimental.pallas import tpu as pltpu
from jax.experimental.pallas import tpu_sc as plsc

N_ENTITIES = 1000000
N_RELATIONS = 1000
EMBED_DIM = 64
BATCH = 16384

NC = 2
NS = 16
NW = NC * NS
B_PER_W = BATCH // NW
CHUNK = 16
NCH = B_PER_W // CHUNK
NBUF = 2
NLANE = 16


def _sc_kernel(heads_hbm, rels_hbm, tails_hbm, ent_hbm, rel_hbm, out_hbm,
               hidx, ridx, tidx, htile, rtile, ttile, hsub, rsub, tsub,
               ebufs, rbufs, tbufs, outb, sems):
    wid = lax.axis_index("s") * NC + lax.axis_index("c")
    base = wid * B_PER_W

    ent_v = ent_hbm.reshape(N_ENTITIES // 8, 8, EMBED_DIM)
    rel_v = rel_hbm.reshape(N_RELATIONS // 8, 8, EMBED_DIM)

    pltpu.sync_copy(heads_hbm.at[pl.ds(base, B_PER_W)], hidx)
    pltpu.sync_copy(rels_hbm.at[pl.ds(base, B_PER_W)], ridx)
    pltpu.sync_copy(tails_hbm.at[pl.ds(base, B_PER_W)], tidx)
    for g in range(B_PER_W // NLANE):
        sl = pl.ds(g * NLANE, NLANE)
        for raw, tile, sub in ((hidx, htile, hsub), (ridx, rtile, rsub),
                               (tidx, ttile, tsub)):
            v = raw[sl]
            tile[sl] = lax.shift_right_logical(v, 3)
            sub[sl] = lax.bitwise_and(v, 7)

    lane = lax.iota(jnp.int32, NLANE)

    def fire(k, slot):
        kk = jnp.minimum(k, NCH - 1)
        isl = pl.ds(kk * CHUNK, CHUNK)
        thv, trv, ttv = htile[isl], rtile[isl], ttile[isl]
        copies = []
        for i in range(CHUNK):
            copies.append(pltpu.async_copy(
                ent_v.at[thv[i]], ebufs[slot].at[i], sems[slot]))
            copies.append(pltpu.async_copy(
                ent_v.at[ttv[i]], tbufs[slot].at[i], sems[slot]))
            copies.append(pltpu.async_copy(
                rel_v.at[trv[i]], rbufs[slot].at[i], sems[slot]))
        return copies

    def wait(slot):
        for i in range(CHUNK):
            pltpu.make_async_copy(ent_v.at[0], ebufs[slot].at[i],
                                  sems[slot]).wait()
            pltpu.make_async_copy(ent_v.at[0], tbufs[slot].at[i],
                                  sems[slot]).wait()
            pltpu.make_async_copy(rel_v.at[0], rbufs[slot].at[i],
                                  sems[slot]).wait()

    def compute(k, slot):
        eb, rb, tb = ebufs[slot], rbufs[slot], tbufs[slot]
        isl = pl.ds(k * CHUNK, NLANE)
        hs, rs, ts = hsub[isl], rsub[isl], tsub[isl]
        out16 = jnp.zeros((NLANE,), jnp.float32)
        for i in range(CHUNK):
            sh, sr, st = hs[i], rs[i], ts[i]
            acc = None
            for c in range(EMBED_DIM // NLANE):
                sl = pl.ds(c * NLANE, NLANE)
                v = eb[i, sh, sl] + rb[i, sr, sl] - tb[i, st, sl]
                acc = v * v if acc is None else acc + v * v
            out16 = jnp.where(lane == i, -jnp.sum(acc), out16)
        outb[pl.ds(k * CHUNK, CHUNK)] = out16

    fire(0, 0)

    def body(g, carry):
        for b in range(NBUF):
            k = g + b
            fire(k + 1, 1 - b)
            wait(b)
            compute(k, b)
        return carry

    lax.fori_loop(0, NCH // NBUF, lambda g, c: body(g * NBUF, c), 0)
    wait(0)

    pltpu.sync_copy(outb, out_hbm.at[pl.ds(base, B_PER_W)])


@jax.jit
def kernel(heads, relations, tails, entity_embedding, relation_embedding):
    mesh = plsc.VectorSubcoreMesh(core_axis_name="c", subcore_axis_name="s")
    k = functools.partial(
        pl.kernel,
        mesh=mesh,
        out_type=jax.ShapeDtypeStruct((BATCH,), jnp.float32),
        compiler_params=pltpu.CompilerParams(
            needs_layout_passes=False, use_tc_tiling_on_sc=True),
        scratch_types=[
            pltpu.VMEM((B_PER_W,), jnp.int32),
            pltpu.VMEM((B_PER_W,), jnp.int32),
            pltpu.VMEM((B_PER_W,), jnp.int32),
            pltpu.VMEM((B_PER_W,), jnp.int32),
            pltpu.VMEM((B_PER_W,), jnp.int32),
            pltpu.VMEM((B_PER_W,), jnp.int32),
            pltpu.VMEM((B_PER_W,), jnp.int32),
            pltpu.VMEM((B_PER_W,), jnp.int32),
            pltpu.VMEM((B_PER_W,), jnp.int32),
            [pltpu.VMEM((CHUNK, 8, EMBED_DIM), jnp.float32) for _ in range(NBUF)],
            [pltpu.VMEM((CHUNK, 8, EMBED_DIM), jnp.float32) for _ in range(NBUF)],
            [pltpu.VMEM((CHUNK, 8, EMBED_DIM), jnp.float32) for _ in range(NBUF)],
            pltpu.VMEM((B_PER_W,), jnp.float32),
            [pltpu.SemaphoreType.DMA for _ in range(NBUF)],
        ],
    )(_sc_kernel)
    return k(heads, relations, tails, entity_embedding, relation_embedding)
